# baseline (speedup 1.0000x reference)
import functools

import jax
import jax.numpy as jnp
from jax import lax
from jax.experimental import pallas as pl
from jax.experimental.pallas import tpu as pltpu

_NP = 20000
_NPP = 20480
_CH = 256
_NCH = _NPP // _CH
_NT = 1000
_TB = 8
_NBLK = _NT // _TB

_BIG = float(2.0 ** 30)


def _body(tref, pref, iou_ref, lab_ref, scr):
    @pl.when(pl.program_id(0) == 0)
    def _init():
        score = pref[6]
        valid = score > 0.5
        scr[0:8, :] = jnp.where(valid, score, jnp.inf)
        vp = (((pref[3] - pref[0] + 1.0) * (pref[4] - pref[1] + 1.0))
              * (pref[5] - pref[2] + 1.0))
        scr[8:16, :] = jnp.where(valid, vp, jnp.inf)
        scr[16:24, :] = jax.lax.broadcasted_iota(
            jnp.int32, (8, _NPP), 1).astype(jnp.float32)

    t = tref[0]
    ttl = [t[:, d:d + 1] for d in range(3)]
    tbr = [t[:, 3 + d:4 + d] for d in range(3)]
    vt = ((tbr[0] - ttl[0] + 1.0) * (tbr[1] - ttl[1] + 1.0)
          * (tbr[2] - ttl[2] + 1.0))

    def chunk(c, carry):
        bv, bk, bj, bl = carry
        s = c * _CH
        p = [pref[f, :, pl.ds(s, _CH)] for f in range(6)]
        key = scr[0:8, pl.ds(s, _CH)]
        vp = scr[8:16, pl.ds(s, _CH)]
        jv = scr[16:24, pl.ds(s, _CH)]
        lb = pref[7, :, pl.ds(s, _CH)]

        e0 = jnp.minimum(tbr[0], p[3]) - jnp.maximum(ttl[0], p[0]) + 1.0
        e1 = jnp.minimum(tbr[1], p[4]) - jnp.maximum(ttl[1], p[1]) + 1.0
        e2 = jnp.minimum(tbr[2], p[5]) - jnp.maximum(ttl[2], p[2]) + 1.0
        inter = (e0 * e1) * e2
        union = (vt + vp) - inter

        ov1 = (tbr[0] > p[0]) | (tbr[1] > p[1]) | (tbr[2] > p[2])
        ov2 = (ttl[0] < p[3]) | (ttl[1] < p[4]) | (ttl[2] < p[5])
        keep = ov1 & ov2

        val = jnp.where(keep, inter / union, 0.0)

        upd = (val > bv) | ((val == bv) & (key < bk))
        bv = jnp.where(upd, val, bv)
        bk = jnp.where(upd, key, bk)
        bj = jnp.where(upd, jv, bj)
        bl = jnp.where(upd, lb, bl)
        return bv, bk, bj, bl

    full = functools.partial(jnp.full, (_TB, _CH), dtype=jnp.float32)
    bv, bk, bj, bl = lax.fori_loop(
        0, _NCH, chunk,
        (full(-jnp.inf), full(jnp.inf), full(_BIG), full(0.0)))

    bm = jnp.max(bv, axis=1, keepdims=True)
    tie = bv == bm
    km = jnp.min(jnp.where(tie, bk, jnp.inf), axis=1, keepdims=True)
    tie2 = tie & (bk == km)
    jm = jnp.min(jnp.where(tie2, bj, _BIG), axis=1, keepdims=True)
    tie3 = tie2 & (bj == jm)
    lab = jnp.max(jnp.where(tie3, bl, -jnp.inf), axis=1, keepdims=True)

    iou_ref[0] = bm
    lab_ref[0] = lab


@jax.jit
def _run(tgt, preds):
    out = pl.pallas_call(
        _body,
        grid=(_NBLK,),
        in_specs=[
            pl.BlockSpec((1, _TB, 8), lambda i: (i, 0, 0)),
            pl.BlockSpec((8, 8, _NPP), lambda i: (0, 0, 0)),
        ],
        out_specs=[
            pl.BlockSpec((1, _TB, 1), lambda i: (i, 0, 0)),
            pl.BlockSpec((1, _TB, 1), lambda i: (i, 0, 0)),
        ],
        out_shape=[
            jax.ShapeDtypeStruct((_NBLK, _TB, 1), jnp.float32),
            jax.ShapeDtypeStruct((_NBLK, _TB, 1), jnp.float32),
        ],
        scratch_shapes=[pltpu.VMEM((24, _NPP), jnp.float32)],
    )(tgt, preds)
    return out


def kernel(pred_boxes, pred_scores, pred_labels, target_boxes, target_labels):
    preds = jnp.concatenate(
        [pred_boxes, pred_scores[:, None], pred_labels[:, None]], axis=1).T
    preds = jnp.pad(preds, ((0, 0), (0, _NPP - _NP)))
    preds = jnp.broadcast_to(preds[:, None, :], (8, 8, _NPP))
    tgt = jnp.concatenate(
        [target_boxes, jnp.zeros((_NT, 2), jnp.float32)], axis=1)
    tgt = tgt.reshape(_NBLK, _TB, 8)
    iou3, lab3 = _run(tgt, preds)
    true_ious = iou3.reshape(_NT)
    pcp_best = lab3.reshape(_NT)
    hit = true_ious > 0.5
    return true_ious, pcp_best, hit, target_labels


# speedup vs baseline: 3.8069x; 3.8069x over previous
import functools

import jax
import jax.numpy as jnp
from jax import lax
from jax.experimental import pallas as pl
from jax.experimental.pallas import tpu as pltpu

_NP = 20000
_NPP = 20480
_CH = 256
_NCH = _NPP // _CH
_NT = 1000
_TB = 8
_NBLK = _NT // _TB

_BIG = float(2.0 ** 30)


def _body(tref, pref, iou_ref, lab_ref, scr):
    @pl.when(pl.program_id(0) == 0)
    def _init():
        score = pref[6]
        valid = score > 0.5
        scr[0:8, :] = jnp.where(valid, score, jnp.inf)
        vp = (((pref[3] - pref[0] + 1.0) * (pref[4] - pref[1] + 1.0))
              * (pref[5] - pref[2] + 1.0))
        scr[8:16, :] = jnp.where(valid, vp, jnp.inf)
        scr[16:24, :] = jax.lax.broadcasted_iota(
            jnp.int32, (8, _NPP), 1).astype(jnp.float32)

    t = tref[0]
    ttl = [t[:, d:d + 1] for d in range(3)]
    tbr = [t[:, 3 + d:4 + d] for d in range(3)]
    vt = ((tbr[0] - ttl[0] + 1.0) * (tbr[1] - ttl[1] + 1.0)
          * (tbr[2] - ttl[2] + 1.0))

    def chunk(c, carry):
        bv, bk, bj, bl = carry
        s = c * _CH
        p = [pref[f, :, pl.ds(s, _CH)] for f in range(6)]
        key = scr[0:8, pl.ds(s, _CH)]
        vp = scr[8:16, pl.ds(s, _CH)]
        jv = scr[16:24, pl.ds(s, _CH)]
        lb = pref[7, :, pl.ds(s, _CH)]

        e0 = jnp.minimum(tbr[0], p[3]) - jnp.maximum(ttl[0], p[0]) + 1.0
        e1 = jnp.minimum(tbr[1], p[4]) - jnp.maximum(ttl[1], p[1]) + 1.0
        e2 = jnp.minimum(tbr[2], p[5]) - jnp.maximum(ttl[2], p[2]) + 1.0
        inter = (e0 * e1) * e2
        union = (vt + vp) - inter

        ov1 = (tbr[0] > p[0]) | (tbr[1] > p[1]) | (tbr[2] > p[2])
        ov2 = (ttl[0] < p[3]) | (ttl[1] < p[4]) | (ttl[2] < p[5])
        keep = ov1 & ov2

        val = jnp.where(keep, inter / union, 0.0)

        upd = (val > bv) | ((val == bv) & (key < bk))
        bv = jnp.where(upd, val, bv)
        bk = jnp.where(upd, key, bk)
        bj = jnp.where(upd, jv, bj)
        bl = jnp.where(upd, lb, bl)
        return bv, bk, bj, bl

    full = functools.partial(jnp.full, (_TB, _CH), dtype=jnp.float32)
    bv, bk, bj, bl = lax.fori_loop(
        0, _NCH, chunk,
        (full(-jnp.inf), full(jnp.inf), full(_BIG), full(0.0)),
        unroll=8)

    bm = jnp.max(bv, axis=1, keepdims=True)
    tie = bv == bm
    km = jnp.min(jnp.where(tie, bk, jnp.inf), axis=1, keepdims=True)
    tie2 = tie & (bk == km)
    jm = jnp.min(jnp.where(tie2, bj, _BIG), axis=1, keepdims=True)
    tie3 = tie2 & (bj == jm)
    lab = jnp.max(jnp.where(tie3, bl, -jnp.inf), axis=1, keepdims=True)

    iou_ref[0] = bm
    lab_ref[0] = lab


@jax.jit
def _run(tgt, preds):
    out = pl.pallas_call(
        _body,
        grid=(_NBLK,),
        in_specs=[
            pl.BlockSpec((1, _TB, 8), lambda i: (i, 0, 0)),
            pl.BlockSpec((8, 8, _NPP), lambda i: (0, 0, 0)),
        ],
        out_specs=[
            pl.BlockSpec((1, _TB, 1), lambda i: (i, 0, 0)),
            pl.BlockSpec((1, _TB, 1), lambda i: (i, 0, 0)),
        ],
        out_shape=[
            jax.ShapeDtypeStruct((_NBLK, _TB, 1), jnp.float32),
            jax.ShapeDtypeStruct((_NBLK, _TB, 1), jnp.float32),
        ],
        scratch_shapes=[pltpu.VMEM((24, _NPP), jnp.float32)],
    )(tgt, preds)
    return out


def kernel(pred_boxes, pred_scores, pred_labels, target_boxes, target_labels):
    preds = jnp.concatenate(
        [pred_boxes, pred_scores[:, None], pred_labels[:, None]], axis=1).T
    preds = jnp.pad(preds, ((0, 0), (0, _NPP - _NP)))
    preds = jnp.broadcast_to(preds[:, None, :], (8, 8, _NPP))
    tgt = jnp.concatenate(
        [target_boxes, jnp.zeros((_NT, 2), jnp.float32)], axis=1)
    tgt = tgt.reshape(_NBLK, _TB, 8)
    iou3, lab3 = _run(tgt, preds)
    true_ious = iou3.reshape(_NT)
    pcp_best = lab3.reshape(_NT)
    hit = true_ious > 0.5
    return true_ious, pcp_best, hit, target_labels


# speedup vs baseline: 4.6561x; 1.2231x over previous
import functools

import jax
import jax.numpy as jnp
from jax import lax
from jax.experimental import pallas as pl
from jax.experimental.pallas import tpu as pltpu

_NP = 20000
_NPP = 20480
_CH = 256
_NCH = _NPP // _CH
_NT = 1000
_TB = 8
_NBLK = _NT // _TB

_BIG = float(2.0 ** 30)


def _body(tref, pref, iou_ref, lab_ref, scr):
    @pl.when(pl.program_id(0) == 0)
    def _init():
        score = pref[6]
        valid = score > 0.5
        scr[0:8, :] = jnp.where(valid, score, jnp.inf)
        vp = (((pref[3] - pref[0] + 1.0) * (pref[4] - pref[1] + 1.0))
              * (pref[5] - pref[2] + 1.0))
        scr[8:16, :] = jnp.where(valid, vp, jnp.inf)
        scr[16:24, :] = jax.lax.broadcasted_iota(
            jnp.int32, (8, _NPP), 1).astype(jnp.float32)

    t = tref[0]
    ttl = [t[:, d:d + 1] for d in range(3)]
    tbr = [t[:, 3 + d:4 + d] for d in range(3)]
    vt = ((tbr[0] - ttl[0] + 1.0) * (tbr[1] - ttl[1] + 1.0)
          * (tbr[2] - ttl[2] + 1.0))

    def chunk(c, carry):
        bv, bk, bj, bl = carry
        s = c * _CH
        p = [pref[f, :, pl.ds(s, _CH)] for f in range(6)]
        key = scr[0:8, pl.ds(s, _CH)]
        vp = scr[8:16, pl.ds(s, _CH)]
        jv = scr[16:24, pl.ds(s, _CH)]
        lb = pref[7, :, pl.ds(s, _CH)]

        e0 = jnp.minimum(tbr[0], p[3]) - jnp.maximum(ttl[0], p[0]) + 1.0
        e1 = jnp.minimum(tbr[1], p[4]) - jnp.maximum(ttl[1], p[1]) + 1.0
        e2 = jnp.minimum(tbr[2], p[5]) - jnp.maximum(ttl[2], p[2]) + 1.0
        inter = (e0 * e1) * e2
        union = (vt + vp) - inter

        ov1 = (tbr[0] > p[0]) | (tbr[1] > p[1]) | (tbr[2] > p[2])
        ov2 = (ttl[0] < p[3]) | (ttl[1] < p[4]) | (ttl[2] < p[5])
        keep = ov1 & ov2

        val = jnp.where(keep, inter / union, 0.0)

        upd = (val > bv) | ((val == bv) & (key < bk))
        bv = jnp.where(upd, val, bv)
        bk = jnp.where(upd, key, bk)
        bj = jnp.where(upd, jv, bj)
        bl = jnp.where(upd, lb, bl)
        return bv, bk, bj, bl

    full = functools.partial(jnp.full, (_TB, _CH), dtype=jnp.float32)
    bv, bk, bj, bl = lax.fori_loop(
        0, _NCH, chunk,
        (full(-jnp.inf), full(jnp.inf), full(_BIG), full(0.0)),
        unroll=16)

    bm = jnp.max(bv, axis=1, keepdims=True)
    tie = bv == bm
    km = jnp.min(jnp.where(tie, bk, jnp.inf), axis=1, keepdims=True)
    tie2 = tie & (bk == km)
    jm = jnp.min(jnp.where(tie2, bj, _BIG), axis=1, keepdims=True)
    tie3 = tie2 & (bj == jm)
    lab = jnp.max(jnp.where(tie3, bl, -jnp.inf), axis=1, keepdims=True)

    iou_ref[0] = bm
    lab_ref[0] = lab


@jax.jit
def _run(tgt, preds):
    out = pl.pallas_call(
        _body,
        grid=(_NBLK,),
        in_specs=[
            pl.BlockSpec((1, _TB, 8), lambda i: (i, 0, 0)),
            pl.BlockSpec((8, 8, _NPP), lambda i: (0, 0, 0)),
        ],
        out_specs=[
            pl.BlockSpec((1, _TB, 1), lambda i: (i, 0, 0)),
            pl.BlockSpec((1, _TB, 1), lambda i: (i, 0, 0)),
        ],
        out_shape=[
            jax.ShapeDtypeStruct((_NBLK, _TB, 1), jnp.float32),
            jax.ShapeDtypeStruct((_NBLK, _TB, 1), jnp.float32),
        ],
        scratch_shapes=[pltpu.VMEM((24, _NPP), jnp.float32)],
    )(tgt, preds)
    return out


def kernel(pred_boxes, pred_scores, pred_labels, target_boxes, target_labels):
    preds = jnp.concatenate(
        [pred_boxes, pred_scores[:, None], pred_labels[:, None]], axis=1).T
    preds = jnp.pad(preds, ((0, 0), (0, _NPP - _NP)))
    preds = jnp.broadcast_to(preds[:, None, :], (8, 8, _NPP))
    tgt = jnp.concatenate(
        [target_boxes, jnp.zeros((_NT, 2), jnp.float32)], axis=1)
    tgt = tgt.reshape(_NBLK, _TB, 8)
    iou3, lab3 = _run(tgt, preds)
    true_ious = iou3.reshape(_NT)
    pcp_best = lab3.reshape(_NT)
    hit = true_ious > 0.5
    return true_ious, pcp_best, hit, target_labels


# speedup vs baseline: 5.3413x; 1.1472x over previous
import functools

import jax
import jax.numpy as jnp
from jax import lax
from jax.experimental import pallas as pl
from jax.experimental.pallas import tpu as pltpu

_NP = 20000
_NPP = 20480
_CH = 256
_NCH = _NPP // _CH
_NT = 1000
_NTP = 1008
_TB = 8
_NBLK = _NTP // (2 * _TB)

_BIG = float(2.0 ** 30)


def _tfields(t):
    ttl = [t[:, d:d + 1] for d in range(3)]
    tbr = [t[:, 3 + d:4 + d] for d in range(3)]
    vt = ((tbr[0] - ttl[0] + 1.0) * (tbr[1] - ttl[1] + 1.0)
          * (tbr[2] - ttl[2] + 1.0))
    return ttl, tbr, vt


def _pairval(ttl, tbr, vt, p, vp):
    e0 = jnp.minimum(tbr[0], p[3]) - jnp.maximum(ttl[0], p[0]) + 1.0
    e1 = jnp.minimum(tbr[1], p[4]) - jnp.maximum(ttl[1], p[1]) + 1.0
    e2 = jnp.minimum(tbr[2], p[5]) - jnp.maximum(ttl[2], p[2]) + 1.0
    inter = (e0 * e1) * e2
    union = (vt + vp) - inter
    ov1 = (tbr[0] > p[0]) | (tbr[1] > p[1]) | (tbr[2] > p[2])
    ov2 = (ttl[0] < p[3]) | (ttl[1] < p[4]) | (ttl[2] < p[5])
    return jnp.where(ov1 & ov2, inter / union, 0.0)


def _merge(carry, val, key, jv, lb):
    bv, bk, bj, bl = carry
    upd = (val > bv) | ((val == bv) & (key < bk))
    return (jnp.where(upd, val, bv), jnp.where(upd, key, bk),
            jnp.where(upd, jv, bj), jnp.where(upd, lb, bl))


def _finalize(carry, iou_ref, lab_ref, b):
    bv, bk, bj, bl = carry
    bm = jnp.max(bv, axis=1, keepdims=True)
    tie = bv == bm
    km = jnp.min(jnp.where(tie, bk, jnp.inf), axis=1, keepdims=True)
    tie2 = tie & (bk == km)
    jm = jnp.min(jnp.where(tie2, bj, _BIG), axis=1, keepdims=True)
    tie3 = tie2 & (bj == jm)
    lab = jnp.max(jnp.where(tie3, bl, -jnp.inf), axis=1, keepdims=True)
    iou_ref[b] = bm
    lab_ref[b] = lab


def _body(tref, pref, iou_ref, lab_ref, scr):
    @pl.when(pl.program_id(0) == 0)
    def _init():
        score = pref[6]
        valid = score > 0.5
        scr[0:8, :] = jnp.where(valid, score, jnp.inf)
        vp = (((pref[3] - pref[0] + 1.0) * (pref[4] - pref[1] + 1.0))
              * (pref[5] - pref[2] + 1.0))
        scr[8:16, :] = jnp.where(valid, vp, jnp.inf)
        scr[16:24, :] = jax.lax.broadcasted_iota(
            jnp.int32, (8, _NPP), 1).astype(jnp.float32)

    ta = _tfields(tref[0])
    tb = _tfields(tref[1])

    def chunk(c, carry):
        ca, cb = carry
        s = c * _CH
        p = [pref[f, :, pl.ds(s, _CH)] for f in range(6)]
        key = scr[0:8, pl.ds(s, _CH)]
        vp = scr[8:16, pl.ds(s, _CH)]
        jv = scr[16:24, pl.ds(s, _CH)]
        lb = pref[7, :, pl.ds(s, _CH)]
        ca = _merge(ca, _pairval(*ta, p, vp), key, jv, lb)
        cb = _merge(cb, _pairval(*tb, p, vp), key, jv, lb)
        return ca, cb

    full = functools.partial(jnp.full, (_TB, _CH), dtype=jnp.float32)
    init = (full(-jnp.inf), full(jnp.inf), full(_BIG), full(0.0))
    ca, cb = lax.fori_loop(0, _NCH, chunk, (init, init), unroll=8)

    _finalize(ca, iou_ref, lab_ref, 0)
    _finalize(cb, iou_ref, lab_ref, 1)


@jax.jit
def _run(tgt, preds):
    out = pl.pallas_call(
        _body,
        grid=(_NBLK,),
        in_specs=[
            pl.BlockSpec((2, _TB, 8), lambda i: (i, 0, 0)),
            pl.BlockSpec((8, 8, _NPP), lambda i: (0, 0, 0)),
        ],
        out_specs=[
            pl.BlockSpec((2, _TB, 1), lambda i: (i, 0, 0)),
            pl.BlockSpec((2, _TB, 1), lambda i: (i, 0, 0)),
        ],
        out_shape=[
            jax.ShapeDtypeStruct((2 * _NBLK, _TB, 1), jnp.float32),
            jax.ShapeDtypeStruct((2 * _NBLK, _TB, 1), jnp.float32),
        ],
        scratch_shapes=[pltpu.VMEM((24, _NPP), jnp.float32)],
    )(tgt, preds)
    return out


def kernel(pred_boxes, pred_scores, pred_labels, target_boxes, target_labels):
    preds = jnp.concatenate(
        [pred_boxes, pred_scores[:, None], pred_labels[:, None]], axis=1).T
    preds = jnp.pad(preds, ((0, 0), (0, _NPP - _NP)))
    preds = jnp.broadcast_to(preds[:, None, :], (8, 8, _NPP))
    tgt = jnp.concatenate(
        [target_boxes, jnp.zeros((_NT, 2), jnp.float32)], axis=1)
    tgt = jnp.pad(tgt, ((0, _NTP - _NT), (0, 0)))
    tgt = tgt.reshape(2 * _NBLK, _TB, 8)
    iou3, lab3 = _run(tgt, preds)
    true_ious = iou3.reshape(_NTP)[:_NT]
    pcp_best = lab3.reshape(_NTP)[:_NT]
    hit = true_ious > 0.5
    return true_ious, pcp_best, hit, target_labels


# speedup vs baseline: 6.1210x; 1.1460x over previous
import functools

import jax
import jax.numpy as jnp
from jax import lax
from jax.experimental import pallas as pl
from jax.experimental.pallas import tpu as pltpu

_NP = 20000
_NPP = 20480
_CH = 256
_NCH = _NPP // _CH
_NT = 1000
_NTP = 1008
_TB = 8
_NBLK = _NTP // (2 * _TB)

_BIG = float(2.0 ** 30)


def _tfields(t):
    ttl = [t[:, d:d + 1] for d in range(3)]
    tbr = [t[:, 3 + d:4 + d] for d in range(3)]
    vt = ((tbr[0] - ttl[0] + 1.0) * (tbr[1] - ttl[1] + 1.0)
          * (tbr[2] - ttl[2] + 1.0))
    return ttl, tbr, vt


def _pairval(ttl, tbr, vt, p, vp):
    e0 = jnp.minimum(tbr[0], p[3]) - jnp.maximum(ttl[0], p[0]) + 1.0
    e1 = jnp.minimum(tbr[1], p[4]) - jnp.maximum(ttl[1], p[1]) + 1.0
    e2 = jnp.minimum(tbr[2], p[5]) - jnp.maximum(ttl[2], p[2]) + 1.0
    inter = (e0 * e1) * e2
    union = (vt + vp) - inter
    ov1 = (tbr[0] > p[0]) | (tbr[1] > p[1]) | (tbr[2] > p[2])
    ov2 = (ttl[0] < p[3]) | (ttl[1] < p[4]) | (ttl[2] < p[5])
    return jnp.where(ov1 & ov2, inter / union, 0.0)


def _merge(carry, val, key, jv, lb):
    bv, bk, bj, bl = carry
    upd = (val > bv) | ((val == bv) & (key < bk))
    return (jnp.where(upd, val, bv), jnp.where(upd, key, bk),
            jnp.where(upd, jv, bj), jnp.where(upd, lb, bl))


def _finalize(carry, iou_ref, lab_ref, b):
    bv, bk, bj, bl = carry
    bm = jnp.max(bv, axis=1, keepdims=True)
    tie = bv == bm
    km = jnp.min(jnp.where(tie, bk, jnp.inf), axis=1, keepdims=True)
    tie2 = tie & (bk == km)
    jm = jnp.min(jnp.where(tie2, bj, _BIG), axis=1, keepdims=True)
    tie3 = tie2 & (bj == jm)
    lab = jnp.max(jnp.where(tie3, bl, -jnp.inf), axis=1, keepdims=True)
    iou_ref[b] = bm
    lab_ref[b] = lab


def _body(tref, pref, iou_ref, lab_ref, scr):
    @pl.when(pl.program_id(0) == 0)
    def _init():
        score = pref[6]
        valid = score > 0.5
        scr[0:8, :] = jnp.where(valid, score, jnp.inf)
        vp = (((pref[3] - pref[0] + 1.0) * (pref[4] - pref[1] + 1.0))
              * (pref[5] - pref[2] + 1.0))
        scr[8:16, :] = jnp.where(valid, vp, jnp.inf)
        scr[16:24, :] = jax.lax.broadcasted_iota(
            jnp.int32, (8, _NPP), 1).astype(jnp.float32)

    ta = _tfields(tref[0])
    tb = _tfields(tref[1])

    def chunk(c, carry):
        ca, cb = carry
        s = c * _CH
        p = [pref[f, :, pl.ds(s, _CH)] for f in range(6)]
        key = scr[0:8, pl.ds(s, _CH)]
        vp = scr[8:16, pl.ds(s, _CH)]
        jv = scr[16:24, pl.ds(s, _CH)]
        lb = pref[7, :, pl.ds(s, _CH)]
        ca = _merge(ca, _pairval(*ta, p, vp), key, jv, lb)
        cb = _merge(cb, _pairval(*tb, p, vp), key, jv, lb)
        return ca, cb

    full = functools.partial(jnp.full, (_TB, _CH), dtype=jnp.float32)
    init = (full(-jnp.inf), full(jnp.inf), full(_BIG), full(0.0))
    ca, cb = lax.fori_loop(0, _NCH, chunk, (init, init), unroll=16)

    _finalize(ca, iou_ref, lab_ref, 0)
    _finalize(cb, iou_ref, lab_ref, 1)


@jax.jit
def _run(tgt, preds):
    out = pl.pallas_call(
        _body,
        grid=(_NBLK,),
        in_specs=[
            pl.BlockSpec((2, _TB, 8), lambda i: (i, 0, 0)),
            pl.BlockSpec((8, 8, _NPP), lambda i: (0, 0, 0)),
        ],
        out_specs=[
            pl.BlockSpec((2, _TB, 1), lambda i: (i, 0, 0)),
            pl.BlockSpec((2, _TB, 1), lambda i: (i, 0, 0)),
        ],
        out_shape=[
            jax.ShapeDtypeStruct((2 * _NBLK, _TB, 1), jnp.float32),
            jax.ShapeDtypeStruct((2 * _NBLK, _TB, 1), jnp.float32),
        ],
        scratch_shapes=[pltpu.VMEM((24, _NPP), jnp.float32)],
    )(tgt, preds)
    return out


def kernel(pred_boxes, pred_scores, pred_labels, target_boxes, target_labels):
    preds = jnp.concatenate(
        [pred_boxes, pred_scores[:, None], pred_labels[:, None]], axis=1).T
    preds = jnp.pad(preds, ((0, 0), (0, _NPP - _NP)))
    preds = jnp.broadcast_to(preds[:, None, :], (8, 8, _NPP))
    tgt = jnp.concatenate(
        [target_boxes, jnp.zeros((_NT, 2), jnp.float32)], axis=1)
    tgt = jnp.pad(tgt, ((0, _NTP - _NT), (0, 0)))
    tgt = tgt.reshape(2 * _NBLK, _TB, 8)
    iou3, lab3 = _run(tgt, preds)
    true_ious = iou3.reshape(_NTP)[:_NT]
    pcp_best = lab3.reshape(_NTP)[:_NT]
    hit = true_ious > 0.5
    return true_ious, pcp_best, hit, target_labels


# speedup vs baseline: 6.9611x; 1.1373x over previous
import functools

import jax
import jax.numpy as jnp
from jax import lax
from jax.experimental import pallas as pl
from jax.experimental.pallas import tpu as pltpu

_NP = 20000
_NPP = 20480
_CH = 256
_NCH = _NPP // _CH
_NT = 1000
_NB = 4
_NTP = 1024
_TB = 8
_NBLK = _NTP // (_NB * _TB)

_BIG = float(2.0 ** 30)


def _tfields(t):
    ttl = [t[:, d:d + 1] for d in range(3)]
    tbr = [t[:, 3 + d:4 + d] for d in range(3)]
    vt = ((tbr[0] - ttl[0] + 1.0) * (tbr[1] - ttl[1] + 1.0)
          * (tbr[2] - ttl[2] + 1.0))
    return ttl, tbr, vt


def _pairval(ttl, tbr, vt, p, vp):
    e0 = jnp.minimum(tbr[0], p[3]) - jnp.maximum(ttl[0], p[0]) + 1.0
    e1 = jnp.minimum(tbr[1], p[4]) - jnp.maximum(ttl[1], p[1]) + 1.0
    e2 = jnp.minimum(tbr[2], p[5]) - jnp.maximum(ttl[2], p[2]) + 1.0
    inter = (e0 * e1) * e2
    union = (vt + vp) - inter
    ov1 = (tbr[0] > p[0]) | (tbr[1] > p[1]) | (tbr[2] > p[2])
    ov2 = (ttl[0] < p[3]) | (ttl[1] < p[4]) | (ttl[2] < p[5])
    return jnp.where(ov1 & ov2, inter / union, 0.0)


def _merge(carry, val, key, jv, lb):
    bv, bk, bj, bl = carry
    upd = (val > bv) | ((val == bv) & (key < bk))
    return (jnp.where(upd, val, bv), jnp.where(upd, key, bk),
            jnp.where(upd, jv, bj), jnp.where(upd, lb, bl))


def _finalize(carry, iou_ref, lab_ref, b):
    bv, bk, bj, bl = carry
    bm = jnp.max(bv, axis=1, keepdims=True)
    tie = bv == bm
    km = jnp.min(jnp.where(tie, bk, jnp.inf), axis=1, keepdims=True)
    tie2 = tie & (bk == km)
    jm = jnp.min(jnp.where(tie2, bj, _BIG), axis=1, keepdims=True)
    tie3 = tie2 & (bj == jm)
    lab = jnp.max(jnp.where(tie3, bl, -jnp.inf), axis=1, keepdims=True)
    iou_ref[b] = bm
    lab_ref[b] = lab


def _body(tref, pref, iou_ref, lab_ref, scr):
    @pl.when(pl.program_id(0) == 0)
    def _init():
        score = pref[6]
        valid = score > 0.5
        scr[0:8, :] = jnp.where(valid, score, jnp.inf)
        vp = (((pref[3] - pref[0] + 1.0) * (pref[4] - pref[1] + 1.0))
              * (pref[5] - pref[2] + 1.0))
        scr[8:16, :] = jnp.where(valid, vp, jnp.inf)
        scr[16:24, :] = jax.lax.broadcasted_iota(
            jnp.int32, (8, _NPP), 1).astype(jnp.float32)

    tf = [_tfields(tref[b]) for b in range(_NB)]

    def chunk(c, carry):
        s = c * _CH
        p = [pref[f, :, pl.ds(s, _CH)] for f in range(6)]
        key = scr[0:8, pl.ds(s, _CH)]
        vp = scr[8:16, pl.ds(s, _CH)]
        jv = scr[16:24, pl.ds(s, _CH)]
        lb = pref[7, :, pl.ds(s, _CH)]
        return tuple(
            _merge(carry[b], _pairval(*tf[b], p, vp), key, jv, lb)
            for b in range(_NB))

    full = functools.partial(jnp.full, (_TB, _CH), dtype=jnp.float32)
    init = (full(-jnp.inf), full(jnp.inf), full(_BIG), full(0.0))
    cs = lax.fori_loop(0, _NCH, chunk, (init,) * _NB, unroll=16)

    for b in range(_NB):
        _finalize(cs[b], iou_ref, lab_ref, b)


@jax.jit
def _run(tgt, preds):
    out = pl.pallas_call(
        _body,
        grid=(_NBLK,),
        in_specs=[
            pl.BlockSpec((_NB, _TB, 8), lambda i: (i, 0, 0)),
            pl.BlockSpec((8, 8, _NPP), lambda i: (0, 0, 0)),
        ],
        out_specs=[
            pl.BlockSpec((_NB, _TB, 1), lambda i: (i, 0, 0)),
            pl.BlockSpec((_NB, _TB, 1), lambda i: (i, 0, 0)),
        ],
        out_shape=[
            jax.ShapeDtypeStruct((_NB * _NBLK, _TB, 1), jnp.float32),
            jax.ShapeDtypeStruct((_NB * _NBLK, _TB, 1), jnp.float32),
        ],
        scratch_shapes=[pltpu.VMEM((24, _NPP), jnp.float32)],
    )(tgt, preds)
    return out


def kernel(pred_boxes, pred_scores, pred_labels, target_boxes, target_labels):
    preds = jnp.concatenate(
        [pred_boxes, pred_scores[:, None], pred_labels[:, None]], axis=1).T
    preds = jnp.pad(preds, ((0, 0), (0, _NPP - _NP)))
    preds = jnp.broadcast_to(preds[:, None, :], (8, 8, _NPP))
    tgt = jnp.concatenate(
        [target_boxes, jnp.zeros((_NT, 2), jnp.float32)], axis=1)
    tgt = jnp.pad(tgt, ((0, _NTP - _NT), (0, 0)))
    tgt = tgt.reshape(_NB * _NBLK, _TB, 8)
    iou3, lab3 = _run(tgt, preds)
    true_ious = iou3.reshape(_NTP)[:_NT]
    pcp_best = lab3.reshape(_NTP)[:_NT]
    hit = true_ious > 0.5
    return true_ious, pcp_best, hit, target_labels


# speedup vs baseline: 7.3108x; 1.0502x over previous
import functools

import jax
import jax.numpy as jnp
from jax import lax
from jax.experimental import pallas as pl
from jax.experimental.pallas import tpu as pltpu

_NP = 20000
_NPP = 20480
_CH = 256
_NCH = _NPP // _CH
_NT = 1000
_NB = 8
_NTP = 1024
_TB = 8
_NBLK = _NTP // (_NB * _TB)

_BIG = float(2.0 ** 30)


def _tfields(t):
    ttl = [t[:, d:d + 1] for d in range(3)]
    tbr = [t[:, 3 + d:4 + d] for d in range(3)]
    vt = ((tbr[0] - ttl[0] + 1.0) * (tbr[1] - ttl[1] + 1.0)
          * (tbr[2] - ttl[2] + 1.0))
    return ttl, tbr, vt


def _pairval(ttl, tbr, vt, p, vp):
    e0 = jnp.minimum(tbr[0], p[3]) - jnp.maximum(ttl[0], p[0]) + 1.0
    e1 = jnp.minimum(tbr[1], p[4]) - jnp.maximum(ttl[1], p[1]) + 1.0
    e2 = jnp.minimum(tbr[2], p[5]) - jnp.maximum(ttl[2], p[2]) + 1.0
    inter = (e0 * e1) * e2
    union = (vt + vp) - inter
    ov1 = (tbr[0] > p[0]) | (tbr[1] > p[1]) | (tbr[2] > p[2])
    ov2 = (ttl[0] < p[3]) | (ttl[1] < p[4]) | (ttl[2] < p[5])
    return jnp.where(ov1 & ov2, inter / union, 0.0)


def _merge(carry, val, key, jv, lb):
    bv, bk, bj, bl = carry
    upd = (val > bv) | ((val == bv) & (key < bk))
    return (jnp.where(upd, val, bv), jnp.where(upd, key, bk),
            jnp.where(upd, jv, bj), jnp.where(upd, lb, bl))


def _finalize(carry, iou_ref, lab_ref, b):
    bv, bk, bj, bl = carry
    bm = jnp.max(bv, axis=1, keepdims=True)
    tie = bv == bm
    km = jnp.min(jnp.where(tie, bk, jnp.inf), axis=1, keepdims=True)
    tie2 = tie & (bk == km)
    jm = jnp.min(jnp.where(tie2, bj, _BIG), axis=1, keepdims=True)
    tie3 = tie2 & (bj == jm)
    lab = jnp.max(jnp.where(tie3, bl, -jnp.inf), axis=1, keepdims=True)
    iou_ref[b] = bm
    lab_ref[b] = lab


def _body(tref, pref, iou_ref, lab_ref, scr):
    @pl.when(pl.program_id(0) == 0)
    def _init():
        score = pref[6]
        valid = score > 0.5
        scr[0:8, :] = jnp.where(valid, score, jnp.inf)
        vp = (((pref[3] - pref[0] + 1.0) * (pref[4] - pref[1] + 1.0))
              * (pref[5] - pref[2] + 1.0))
        scr[8:16, :] = jnp.where(valid, vp, jnp.inf)
        scr[16:24, :] = jax.lax.broadcasted_iota(
            jnp.int32, (8, _NPP), 1).astype(jnp.float32)

    tf = [_tfields(tref[b]) for b in range(_NB)]

    def chunk(c, carry):
        s = c * _CH
        p = [pref[f, :, pl.ds(s, _CH)] for f in range(6)]
        key = scr[0:8, pl.ds(s, _CH)]
        vp = scr[8:16, pl.ds(s, _CH)]
        jv = scr[16:24, pl.ds(s, _CH)]
        lb = pref[7, :, pl.ds(s, _CH)]
        return tuple(
            _merge(carry[b], _pairval(*tf[b], p, vp), key, jv, lb)
            for b in range(_NB))

    full = functools.partial(jnp.full, (_TB, _CH), dtype=jnp.float32)
    init = (full(-jnp.inf), full(jnp.inf), full(_BIG), full(0.0))
    cs = lax.fori_loop(0, _NCH, chunk, (init,) * _NB, unroll=8)

    for b in range(_NB):
        _finalize(cs[b], iou_ref, lab_ref, b)


@jax.jit
def _run(tgt, preds):
    out = pl.pallas_call(
        _body,
        grid=(_NBLK,),
        in_specs=[
            pl.BlockSpec((_NB, _TB, 8), lambda i: (i, 0, 0)),
            pl.BlockSpec((8, 8, _NPP), lambda i: (0, 0, 0)),
        ],
        out_specs=[
            pl.BlockSpec((_NB, _TB, 1), lambda i: (i, 0, 0)),
            pl.BlockSpec((_NB, _TB, 1), lambda i: (i, 0, 0)),
        ],
        out_shape=[
            jax.ShapeDtypeStruct((_NB * _NBLK, _TB, 1), jnp.float32),
            jax.ShapeDtypeStruct((_NB * _NBLK, _TB, 1), jnp.float32),
        ],
        scratch_shapes=[pltpu.VMEM((24, _NPP), jnp.float32)],
    )(tgt, preds)
    return out


def kernel(pred_boxes, pred_scores, pred_labels, target_boxes, target_labels):
    preds = jnp.concatenate(
        [pred_boxes, pred_scores[:, None], pred_labels[:, None]], axis=1).T
    preds = jnp.pad(preds, ((0, 0), (0, _NPP - _NP)))
    preds = jnp.broadcast_to(preds[:, None, :], (8, 8, _NPP))
    tgt = jnp.concatenate(
        [target_boxes, jnp.zeros((_NT, 2), jnp.float32)], axis=1)
    tgt = jnp.pad(tgt, ((0, _NTP - _NT), (0, 0)))
    tgt = tgt.reshape(_NB * _NBLK, _TB, 8)
    iou3, lab3 = _run(tgt, preds)
    true_ious = iou3.reshape(_NTP)[:_NT]
    pcp_best = lab3.reshape(_NTP)[:_NT]
    hit = true_ious > 0.5
    return true_ious, pcp_best, hit, target_labels


# speedup vs baseline: 7.5707x; 1.0356x over previous
import functools

import jax
import jax.numpy as jnp
from jax import lax
from jax.experimental import pallas as pl
from jax.experimental.pallas import tpu as pltpu

_NP = 20000
_NPP = 20480
_CH = 256
_NCH = _NPP // _CH
_NT = 1000
_NB = 8
_NTP = 1024
_TB = 8
_NBLK = _NTP // (_NB * _TB)

_BIG = float(2.0 ** 30)


def _tfields(t):
    ttl = [t[:, d:d + 1] for d in range(3)]
    tbr = [t[:, 3 + d:4 + d] for d in range(3)]
    vt = ((tbr[0] - ttl[0] + 1.0) * (tbr[1] - ttl[1] + 1.0)
          * (tbr[2] - ttl[2] + 1.0))
    return ttl, tbr, vt


def _pairval(ttl, tbr, vt, p, vp):
    e0 = jnp.minimum(tbr[0], p[3]) - jnp.maximum(ttl[0], p[0]) + 1.0
    e1 = jnp.minimum(tbr[1], p[4]) - jnp.maximum(ttl[1], p[1]) + 1.0
    e2 = jnp.minimum(tbr[2], p[5]) - jnp.maximum(ttl[2], p[2]) + 1.0
    inter = (e0 * e1) * e2
    union = (vt + vp) - inter
    ov1 = (tbr[0] > p[0]) | (tbr[1] > p[1]) | (tbr[2] > p[2])
    ov2 = (ttl[0] < p[3]) | (ttl[1] < p[4]) | (ttl[2] < p[5])
    return jnp.where(ov1 & ov2, inter / union, 0.0)


def _merge(carry, val, key, jv, lb):
    bv, bk, bj, bl = carry
    upd = (val > bv) | ((val == bv) & (key < bk))
    return (jnp.where(upd, val, bv), jnp.where(upd, key, bk),
            jnp.where(upd, jv, bj), jnp.where(upd, lb, bl))


def _finalize(carry, iou_ref, lab_ref, b):
    bv, bk, bj, bl = carry
    bm = jnp.max(bv, axis=1, keepdims=True)
    tie = bv == bm
    km = jnp.min(jnp.where(tie, bk, jnp.inf), axis=1, keepdims=True)
    tie2 = tie & (bk == km)
    jm = jnp.min(jnp.where(tie2, bj, _BIG), axis=1, keepdims=True)
    tie3 = tie2 & (bj == jm)
    lab = jnp.max(jnp.where(tie3, bl, -jnp.inf), axis=1, keepdims=True)
    iou_ref[b] = bm
    lab_ref[b] = lab


def _body(tref, pref, iou_ref, lab_ref, scr):
    @pl.when(pl.program_id(0) == 0)
    def _init():
        score = pref[6]
        valid = score > 0.5
        scr[0:8, :] = jnp.where(valid, score, jnp.inf)
        vp = (((pref[3] - pref[0] + 1.0) * (pref[4] - pref[1] + 1.0))
              * (pref[5] - pref[2] + 1.0))
        scr[8:16, :] = jnp.where(valid, vp, jnp.inf)
        scr[16:24, :] = jax.lax.broadcasted_iota(
            jnp.int32, (8, _NPP), 1).astype(jnp.float32)

    tf = [_tfields(tref[b]) for b in range(_NB)]

    def chunk(c, carry):
        s = c * _CH
        p = [pref[f, :, pl.ds(s, _CH)] for f in range(6)]
        key = scr[0:8, pl.ds(s, _CH)]
        vp = scr[8:16, pl.ds(s, _CH)]
        jv = scr[16:24, pl.ds(s, _CH)]
        lb = pref[7, :, pl.ds(s, _CH)]
        return tuple(
            _merge(carry[b], _pairval(*tf[b], p, vp), key, jv, lb)
            for b in range(_NB))

    full = functools.partial(jnp.full, (_TB, _CH), dtype=jnp.float32)
    init = (full(-jnp.inf), full(jnp.inf), full(_BIG), full(0.0))
    cs = lax.fori_loop(0, _NCH, chunk, (init,) * _NB, unroll=16)

    for b in range(_NB):
        _finalize(cs[b], iou_ref, lab_ref, b)


@jax.jit
def _run(tgt, preds):
    out = pl.pallas_call(
        _body,
        grid=(_NBLK,),
        in_specs=[
            pl.BlockSpec((_NB, _TB, 8), lambda i: (i, 0, 0)),
            pl.BlockSpec((8, 8, _NPP), lambda i: (0, 0, 0)),
        ],
        out_specs=[
            pl.BlockSpec((_NB, _TB, 1), lambda i: (i, 0, 0)),
            pl.BlockSpec((_NB, _TB, 1), lambda i: (i, 0, 0)),
        ],
        out_shape=[
            jax.ShapeDtypeStruct((_NB * _NBLK, _TB, 1), jnp.float32),
            jax.ShapeDtypeStruct((_NB * _NBLK, _TB, 1), jnp.float32),
        ],
        scratch_shapes=[pltpu.VMEM((24, _NPP), jnp.float32)],
    )(tgt, preds)
    return out


def kernel(pred_boxes, pred_scores, pred_labels, target_boxes, target_labels):
    preds = jnp.concatenate(
        [pred_boxes, pred_scores[:, None], pred_labels[:, None]], axis=1).T
    preds = jnp.pad(preds, ((0, 0), (0, _NPP - _NP)))
    preds = jnp.broadcast_to(preds[:, None, :], (8, 8, _NPP))
    tgt = jnp.concatenate(
        [target_boxes, jnp.zeros((_NT, 2), jnp.float32)], axis=1)
    tgt = jnp.pad(tgt, ((0, _NTP - _NT), (0, 0)))
    tgt = tgt.reshape(_NB * _NBLK, _TB, 8)
    iou3, lab3 = _run(tgt, preds)
    true_ious = iou3.reshape(_NTP)[:_NT]
    pcp_best = lab3.reshape(_NTP)[:_NT]
    hit = true_ious > 0.5
    return true_ious, pcp_best, hit, target_labels
